# baseline (speedup 1.0000x reference)
import functools

import jax
import jax.numpy as jnp
from jax import lax
from jax.experimental import pallas as pl
from jax.experimental.pallas import tpu as pltpu
from jax.experimental.pallas import tpu_sc as plsc

N = 10000
NP = 10112
E = 320000
D = 128
NC = 2
NS = 16
L = 16
NW = NC * NS
CH = 128
CPW = 80
EPT = CPW * CH
E_PAD = NW * EPT
PAD = E_PAD - E
NB = 2
ROWS_PT = NP // NS


def _worker_id():
    return lax.axis_index("s") * NC + lax.axis_index("c")


def _mesh():
    return plsc.VectorSubcoreMesh(
        core_axis_name="c", subcore_axis_name="s", num_cores=NC, num_subcores=NS
    )


def _degree_body(src_hbm, dst_hbm, dego_hbm, degi_hbm,
                 dego_loc, degi_loc, sv, dv):
    wid = _worker_id()
    base_e = wid * EPT
    pltpu.sync_copy(src_hbm.at[pl.ds(base_e, EPT)], sv)
    pltpu.sync_copy(dst_hbm.at[pl.ds(base_e, EPT)], dv)

    def z(i, _):
        dego_loc[pl.ds(i * L, L)] = jnp.zeros((L,), jnp.float32)
        degi_loc[pl.ds(i * L, L)] = jnp.zeros((L,), jnp.float32)
        return 0

    lax.fori_loop(0, N // L, z, 0)

    ones = jnp.ones((L,), jnp.float32)

    def grp(g, _):
        plsc.addupdate_scatter(dego_loc, [sv[pl.ds(g * L, L)]], ones)
        plsc.addupdate_scatter(degi_loc, [dv[pl.ds(g * L, L)]], ones)
        return 0

    lax.fori_loop(0, EPT // L, grp, 0)

    pltpu.sync_copy(dego_loc, dego_hbm.at[pl.ds(wid * N, N)])
    pltpu.sync_copy(degi_loc, degi_hbm.at[pl.ds(wid * N, N)])


def _edge_weight_body(src_hbm, dst_hbm, ew_hbm, no_hbm, ni_hbm, we_hbm,
                      no_v, ni_v, sv, dv, ewv, ov):
    wid = _worker_id()
    base_e = wid * EPT
    pltpu.sync_copy(no_hbm, no_v)
    pltpu.sync_copy(ni_hbm, ni_v)
    pltpu.sync_copy(src_hbm.at[pl.ds(base_e, EPT)], sv)
    pltpu.sync_copy(dst_hbm.at[pl.ds(base_e, EPT)], dv)
    pltpu.sync_copy(ew_hbm.at[pl.ds(base_e, EPT)], ewv)

    def grp(g, _):
        s = sv[pl.ds(g * L, L)]
        d = dv[pl.ds(g * L, L)]
        w = ewv[pl.ds(g * L, L)]
        ov[pl.ds(g * L, L)] = (
            w * plsc.load_gather(no_v, [s]) * plsc.load_gather(ni_v, [d])
        )
        return 0

    lax.fori_loop(0, EPT // L, grp, 0)
    pltpu.sync_copy(ov, we_hbm.at[pl.ds(base_e, EPT)])


def _spmm_body(x_hbm, src_hbm, dst2d_hbm, we_hbm, zeros_hbm, acc_hbm,
               dv2, sv0, sv1, wv0, wv1, r0, r1,
               i0, i1, g0, g1, s0, s1, acc_sh):
    cid = lax.axis_index("c")
    sid = lax.axis_index("s")
    wid = _worker_id()
    svb = (sv0, sv1)
    wvb = (wv0, wv1)
    rows = (r0, r1)
    isem = (i0, i1)
    gsem = (g0, g1)
    ssem = (s0, s1)

    base_e = wid * EPT
    pltpu.sync_copy(dst2d_hbm.at[pl.ds(wid * CPW, CPW)], dv2)

    base_r = sid * ROWS_PT
    pltpu.sync_copy(zeros_hbm.at[pl.ds(base_r, ROWS_PT)],
                    acc_sh.at[pl.ds(base_r, ROWS_PT)])
    plsc.subcore_barrier()

    def idx_load(i, b):
        off = base_e + i * CH
        pltpu.async_copy(src_hbm.at[pl.ds(off, CH)], svb[b], isem[b])
        pltpu.async_copy(we_hbm.at[pl.ds(off, CH)], wvb[b], isem[b])

    def wait_idx(b):
        pltpu.make_async_copy(
            src_hbm.at[pl.ds(0, CH)], svb[b], isem[b]).wait()
        pltpu.make_async_copy(
            we_hbm.at[pl.ds(0, CH)], wvb[b], isem[b]).wait()

    def issue_gather(b):
        pltpu.async_copy(x_hbm.at[svb[b]], rows[b], gsem[b])

    def wait_gather(b):
        pltpu.make_async_copy(x_hbm.at[svb[b]], rows[b], gsem[b]).wait()

    def issue_scatter(i, b):
        pltpu.async_copy(rows[b], acc_sh.at[dv2.at[i]], ssem[b], add=True)

    def wait_scatter(b):
        pltpu.make_async_copy(rows[b], acc_sh.at[dv2.at[0]], ssem[b]).wait()

    idx_load(0, 0)
    wait_idx(0)
    issue_gather(0)
    idx_load(1, 1)

    def outer(j, _):
        for b in range(NB):
            i = j * NB + b
            wait_gather(b)

            f = i + 1
            q = 1 - b

            @pl.when(f < CPW)
            def _():
                wait_idx(q)

                @pl.when(f >= NB)
                def _():
                    wait_scatter(q)

                issue_gather(q)

            @plsc.parallel_loop(0, CH, unroll=8)
            def scale(jj, _b=b):
                wb = plsc.load_gather(wvb[_b], [jnp.zeros((L,), jnp.int32) + jj])
                for k in range(D // L):
                    rows[_b][jj, pl.ds(k * L, L)] = (
                        rows[_b][jj, pl.ds(k * L, L)] * wb)

            issue_scatter(i, b)

            g = i + 2

            @pl.when(g < CPW)
            def _():
                idx_load(g, b)

        return 0

    lax.fori_loop(0, CPW // NB, outer, 0)
    for b in range(NB):
        wait_scatter(b)

    plsc.subcore_barrier()
    pltpu.sync_copy(acc_sh.at[pl.ds(base_r, ROWS_PT)],
                    acc_hbm.at[cid, pl.ds(base_r, ROWS_PT)])


@functools.lru_cache(maxsize=None)
def _sc_kernels():
    mesh = _mesh()
    params = pltpu.CompilerParams(needs_layout_passes=False)
    degree = pl.kernel(
        _degree_body,
        out_type=(
            jax.ShapeDtypeStruct((NW * N,), jnp.float32),
            jax.ShapeDtypeStruct((NW * N,), jnp.float32),
        ),
        mesh=mesh,
        compiler_params=params,
        scratch_types=(
            pltpu.VMEM((N,), jnp.float32),
            pltpu.VMEM((N,), jnp.float32),
            pltpu.VMEM((EPT,), jnp.int32),
            pltpu.VMEM((EPT,), jnp.int32),
        ),
    )
    edge_weight = pl.kernel(
        _edge_weight_body,
        out_type=jax.ShapeDtypeStruct((E_PAD,), jnp.float32),
        mesh=mesh,
        compiler_params=params,
        scratch_types=(
            pltpu.VMEM((N,), jnp.float32),
            pltpu.VMEM((N,), jnp.float32),
            pltpu.VMEM((EPT,), jnp.int32),
            pltpu.VMEM((EPT,), jnp.int32),
            pltpu.VMEM((EPT,), jnp.float32),
            pltpu.VMEM((EPT,), jnp.float32),
        ),
    )
    spmm = pl.kernel(
        _spmm_body,
        out_type=jax.ShapeDtypeStruct((NC, NP, D), jnp.float32),
        mesh=mesh,
        compiler_params=params,
        scratch_types=(
            pltpu.VMEM((CPW, CH), jnp.int32),
            pltpu.VMEM((CH,), jnp.int32),
            pltpu.VMEM((CH,), jnp.int32),
            pltpu.VMEM((CH,), jnp.float32),
            pltpu.VMEM((CH,), jnp.float32),
            pltpu.VMEM((CH, D), jnp.float32),
            pltpu.VMEM((CH, D), jnp.float32),
            pltpu.SemaphoreType.DMA,
            pltpu.SemaphoreType.DMA,
            pltpu.SemaphoreType.DMA,
            pltpu.SemaphoreType.DMA,
            pltpu.SemaphoreType.DMA,
            pltpu.SemaphoreType.DMA,
            pltpu.VMEM_SHARED((NP, D), jnp.float32),
        ),
    )
    return degree, edge_weight, spmm


def _matmul(accp, Wm, bv, act):
    RB = 1000

    def body(a_ref, w_ref, b_ref, o_ref):
        acc = a_ref[0] + a_ref[1]
        y = jnp.dot(acc, w_ref[...], preferred_element_type=jnp.float32)
        y = y + b_ref[...]
        if act:
            y = jnp.where(y > 0, y, jnp.exp(y) - 1.0)
        o_ref[...] = y

    return pl.pallas_call(
        body,
        grid=(N // RB,),
        in_specs=[
            pl.BlockSpec((NC, RB, D), lambda i: (0, i, 0)),
            pl.BlockSpec((D, D), lambda i: (0, 0)),
            pl.BlockSpec((1, D), lambda i: (0, 0)),
        ],
        out_specs=pl.BlockSpec((RB, D), lambda i: (i, 0)),
        out_shape=jax.ShapeDtypeStruct((N, D), jnp.float32),
    )(accp, Wm, bv)


def kernel(h, edge_index, edge_weight, W0, b0, W1, b1, W2, b2):
    src = edge_index[0]
    dst = edge_index[1]
    pad_idx = jnp.arange(PAD, dtype=src.dtype)
    srcp = jnp.concatenate([src, pad_idx])
    dstp = jnp.concatenate([dst, pad_idx])
    ewp = jnp.concatenate([edge_weight, jnp.zeros((PAD,), edge_weight.dtype)])
    dst2d = dstp.reshape(E_PAD // CH, CH)
    _degree_kernel, _edge_weight_kernel, _spmm_kernel = _sc_kernels()

    pad_corr = (jnp.arange(N) < PAD).astype(jnp.float32)
    dego_p, degi_p = _degree_kernel(srcp, dstp)
    dego = dego_p.reshape(NW, N).sum(axis=0) - pad_corr
    degi = degi_p.reshape(NW, N).sum(axis=0) - pad_corr
    no = jnp.where(dego > 0, dego, 1.0) ** -0.5
    ni = jnp.where(degi > 0, degi, 1.0) ** -0.5

    we = _edge_weight_kernel(srcp, dstp, ewp, no, ni)

    zeros = jnp.zeros((NP, D), jnp.float32)
    x = h
    for Wm, bv, act in ((W0, b0, True), (W1, b1, True), (W2, b2, False)):
        accp = _spmm_kernel(x, srcp, dst2d, we, zeros)
        x = _matmul(accp, Wm, bv.reshape(1, D), act)
    return x


# speedup vs baseline: 1.0083x; 1.0083x over previous
import functools

import jax
import jax.numpy as jnp
from jax import lax
from jax.experimental import pallas as pl
from jax.experimental.pallas import tpu as pltpu
from jax.experimental.pallas import tpu_sc as plsc

N = 10000
NP = 10112
E = 320000
D = 128
NC = 2
NS = 16
L = 16
NW = NC * NS
CH = 128
CPW = 80
EPT = CPW * CH
E_PAD = NW * EPT
PAD = E_PAD - E
NB = 2
ROWS_PT = NP // NS


def _worker_id():
    return lax.axis_index("s") * NC + lax.axis_index("c")


def _mesh():
    return plsc.VectorSubcoreMesh(
        core_axis_name="c", subcore_axis_name="s", num_cores=NC, num_subcores=NS
    )


def _degree_body(src_hbm, dst_hbm, dego_hbm, degi_hbm,
                 dego_loc, degi_loc, sv, dv):
    wid = _worker_id()
    base_e = wid * EPT
    pltpu.sync_copy(src_hbm.at[pl.ds(base_e, EPT)], sv)
    pltpu.sync_copy(dst_hbm.at[pl.ds(base_e, EPT)], dv)

    def z(i, _):
        dego_loc[pl.ds(i * L, L)] = jnp.zeros((L,), jnp.float32)
        degi_loc[pl.ds(i * L, L)] = jnp.zeros((L,), jnp.float32)
        return 0

    lax.fori_loop(0, N // L, z, 0)

    ones = jnp.ones((L,), jnp.float32)

    def grp(g, _):
        plsc.addupdate_scatter(dego_loc, [sv[pl.ds(g * L, L)]], ones)
        plsc.addupdate_scatter(degi_loc, [dv[pl.ds(g * L, L)]], ones)
        return 0

    lax.fori_loop(0, EPT // L, grp, 0)

    pltpu.sync_copy(dego_loc, dego_hbm.at[pl.ds(wid * N, N)])
    pltpu.sync_copy(degi_loc, degi_hbm.at[pl.ds(wid * N, N)])


def _edge_weight_body(src_hbm, dst_hbm, ew_hbm, no_hbm, ni_hbm, we_hbm,
                      no_v, ni_v, sv, dv, ewv, ov):
    wid = _worker_id()
    base_e = wid * EPT
    pltpu.sync_copy(no_hbm, no_v)
    pltpu.sync_copy(ni_hbm, ni_v)
    pltpu.sync_copy(src_hbm.at[pl.ds(base_e, EPT)], sv)
    pltpu.sync_copy(dst_hbm.at[pl.ds(base_e, EPT)], dv)
    pltpu.sync_copy(ew_hbm.at[pl.ds(base_e, EPT)], ewv)

    def grp(g, _):
        s = sv[pl.ds(g * L, L)]
        d = dv[pl.ds(g * L, L)]
        w = ewv[pl.ds(g * L, L)]
        ov[pl.ds(g * L, L)] = (
            w * plsc.load_gather(no_v, [s]) * plsc.load_gather(ni_v, [d])
        )
        return 0

    lax.fori_loop(0, EPT // L, grp, 0)
    pltpu.sync_copy(ov, we_hbm.at[pl.ds(base_e, EPT)])


def _spmm_body(x_hbm, src_hbm, dst2d_hbm, we_hbm, zeros_hbm, acc_hbm,
               dv2, sv0, sv1, wv0, wv1, r0, r1,
               i0, i1, g0, g1, s0, s1, acc_sh):
    cid = lax.axis_index("c")
    sid = lax.axis_index("s")
    wid = _worker_id()
    svb = (sv0, sv1)
    wvb = (wv0, wv1)
    rows = (r0, r1)
    isem = (i0, i1)
    gsem = (g0, g1)
    ssem = (s0, s1)

    base_e = wid * EPT
    pltpu.sync_copy(dst2d_hbm.at[pl.ds(wid * CPW, CPW)], dv2)

    base_r = sid * ROWS_PT
    pltpu.sync_copy(zeros_hbm.at[pl.ds(base_r, ROWS_PT)],
                    acc_sh.at[pl.ds(base_r, ROWS_PT)])
    plsc.subcore_barrier()

    def idx_load(i, b):
        off = base_e + i * CH
        pltpu.async_copy(src_hbm.at[pl.ds(off, CH)], svb[b], isem[b])
        pltpu.async_copy(we_hbm.at[pl.ds(off, CH)], wvb[b], isem[b])

    def wait_idx(b):
        pltpu.make_async_copy(
            src_hbm.at[pl.ds(0, CH)], svb[b], isem[b]).wait()
        pltpu.make_async_copy(
            we_hbm.at[pl.ds(0, CH)], wvb[b], isem[b]).wait()

    def issue_gather(b):
        pltpu.async_copy(x_hbm.at[svb[b]], rows[b], gsem[b])

    def wait_gather(b):
        pltpu.make_async_copy(x_hbm.at[svb[b]], rows[b], gsem[b]).wait()

    def issue_scatter(i, b):
        pltpu.async_copy(rows[b], acc_sh.at[dv2.at[i]], ssem[b], add=True)

    def wait_scatter(b):
        pltpu.make_async_copy(rows[b], acc_sh.at[dv2.at[0]], ssem[b]).wait()

    idx_load(0, 0)
    wait_idx(0)
    issue_gather(0)
    idx_load(1, 1)

    def outer(j, _):
        for b in range(NB):
            i = j * NB + b
            wait_gather(b)

            f = i + 1
            q = 1 - b

            @pl.when(f < CPW)
            def _():
                wait_idx(q)

                @pl.when(f >= NB)
                def _():
                    wait_scatter(q)

                issue_gather(q)

            @plsc.parallel_loop(0, CH, unroll=4)
            def scale(jj, _b=b):
                wb = plsc.load_gather(wvb[_b], [jnp.zeros((L,), jnp.int32) + jj])
                for k in range(D // L):
                    rows[_b][jj, pl.ds(k * L, L)] = (
                        rows[_b][jj, pl.ds(k * L, L)] * wb)

            issue_scatter(i, b)

            g = i + 2

            @pl.when(g < CPW)
            def _():
                idx_load(g, b)

        return 0

    lax.fori_loop(0, CPW // NB, outer, 0)
    for b in range(NB):
        wait_scatter(b)

    plsc.subcore_barrier()
    pltpu.sync_copy(acc_sh.at[pl.ds(base_r, ROWS_PT)],
                    acc_hbm.at[cid, pl.ds(base_r, ROWS_PT)])


@functools.lru_cache(maxsize=None)
def _sc_kernels():
    mesh = _mesh()
    params = pltpu.CompilerParams(needs_layout_passes=False)
    degree = pl.kernel(
        _degree_body,
        out_type=(
            jax.ShapeDtypeStruct((NW * N,), jnp.float32),
            jax.ShapeDtypeStruct((NW * N,), jnp.float32),
        ),
        mesh=mesh,
        compiler_params=params,
        scratch_types=(
            pltpu.VMEM((N,), jnp.float32),
            pltpu.VMEM((N,), jnp.float32),
            pltpu.VMEM((EPT,), jnp.int32),
            pltpu.VMEM((EPT,), jnp.int32),
        ),
    )
    edge_weight = pl.kernel(
        _edge_weight_body,
        out_type=jax.ShapeDtypeStruct((E_PAD,), jnp.float32),
        mesh=mesh,
        compiler_params=params,
        scratch_types=(
            pltpu.VMEM((N,), jnp.float32),
            pltpu.VMEM((N,), jnp.float32),
            pltpu.VMEM((EPT,), jnp.int32),
            pltpu.VMEM((EPT,), jnp.int32),
            pltpu.VMEM((EPT,), jnp.float32),
            pltpu.VMEM((EPT,), jnp.float32),
        ),
    )
    spmm = pl.kernel(
        _spmm_body,
        out_type=jax.ShapeDtypeStruct((NC, NP, D), jnp.float32),
        mesh=mesh,
        compiler_params=params,
        scratch_types=(
            pltpu.VMEM((CPW, CH), jnp.int32),
            pltpu.VMEM((CH,), jnp.int32),
            pltpu.VMEM((CH,), jnp.int32),
            pltpu.VMEM((CH,), jnp.float32),
            pltpu.VMEM((CH,), jnp.float32),
            pltpu.VMEM((CH, D), jnp.float32),
            pltpu.VMEM((CH, D), jnp.float32),
            pltpu.SemaphoreType.DMA,
            pltpu.SemaphoreType.DMA,
            pltpu.SemaphoreType.DMA,
            pltpu.SemaphoreType.DMA,
            pltpu.SemaphoreType.DMA,
            pltpu.SemaphoreType.DMA,
            pltpu.VMEM_SHARED((NP, D), jnp.float32),
        ),
    )
    return degree, edge_weight, spmm


def _matmul(accp, Wm, bv, act):
    RB = 1000

    def body(a_ref, w_ref, b_ref, o_ref):
        acc = a_ref[0] + a_ref[1]
        y = jnp.dot(acc, w_ref[...], preferred_element_type=jnp.float32)
        y = y + b_ref[...]
        if act:
            y = jnp.where(y > 0, y, jnp.exp(y) - 1.0)
        o_ref[...] = y

    return pl.pallas_call(
        body,
        grid=(N // RB,),
        in_specs=[
            pl.BlockSpec((NC, RB, D), lambda i: (0, i, 0)),
            pl.BlockSpec((D, D), lambda i: (0, 0)),
            pl.BlockSpec((1, D), lambda i: (0, 0)),
        ],
        out_specs=pl.BlockSpec((RB, D), lambda i: (i, 0)),
        out_shape=jax.ShapeDtypeStruct((N, D), jnp.float32),
    )(accp, Wm, bv)


def kernel(h, edge_index, edge_weight, W0, b0, W1, b1, W2, b2):
    src = edge_index[0]
    dst = edge_index[1]
    pad_idx = jnp.arange(PAD, dtype=src.dtype)
    srcp = jnp.concatenate([src, pad_idx])
    dstp = jnp.concatenate([dst, pad_idx])
    ewp = jnp.concatenate([edge_weight, jnp.zeros((PAD,), edge_weight.dtype)])
    dst2d = dstp.reshape(E_PAD // CH, CH)
    _degree_kernel, _edge_weight_kernel, _spmm_kernel = _sc_kernels()

    pad_corr = (jnp.arange(N) < PAD).astype(jnp.float32)
    dego_p, degi_p = _degree_kernel(srcp, dstp)
    dego = dego_p.reshape(NW, N).sum(axis=0) - pad_corr
    degi = degi_p.reshape(NW, N).sum(axis=0) - pad_corr
    no = jnp.where(dego > 0, dego, 1.0) ** -0.5
    ni = jnp.where(degi > 0, degi, 1.0) ** -0.5

    we = _edge_weight_kernel(srcp, dstp, ewp, no, ni)

    zeros = jnp.zeros((NP, D), jnp.float32)
    x = h
    for Wm, bv, act in ((W0, b0, True), (W1, b1, True), (W2, b2, False)):
        accp = _spmm_kernel(x, srcp, dst2d, we, zeros)
        x = _matmul(accp, Wm, bv.reshape(1, D), act)
    return x
